# baseline (speedup 1.0000x reference)
import functools

import jax
import jax.numpy as jnp
from jax import lax
from jax.experimental import pallas as pl
from jax.experimental.pallas import tpu as pltpu
from jax.experimental.pallas import tpu_sc as plsc

ROWS = 128
NBUF = 2


@functools.lru_cache(maxsize=None)
def _make_gather(V, D, B):
    info = plsc.get_sparse_core_info()
    NC, NS = info.num_cores, info.num_subcores
    NW = NC * NS
    assert B % (NW * ROWS) == 0
    b_per_w = B // NW
    n_steps = b_per_w // ROWS
    assert n_steps % NBUF == 0
    n_groups = n_steps // NBUF

    mesh = plsc.VectorSubcoreMesh(core_axis_name="c", subcore_axis_name="s")

    @functools.partial(
        pl.kernel,
        mesh=mesh,
        out_type=jax.ShapeDtypeStruct((B, D), jnp.float32),
        compiler_params=pltpu.CompilerParams(use_tc_tiling_on_sc=False),
        scratch_types=[
            pltpu.VMEM((n_steps, ROWS), jnp.int32),
            pltpu.VMEM((NBUF, ROWS, D), jnp.float32),
            pltpu.SemaphoreType.DMA,
            pltpu.SemaphoreType.DMA,
            pltpu.SemaphoreType.DMA,
            pltpu.SemaphoreType.DMA,
        ],
    )
    def gather_kernel(table_hbm, ids_hbm, out_hbm, idx_v, rows_v,
                      gsem0, gsem1, wsem0, wsem1):
        gsem = (gsem0, gsem1)
        wsem = (wsem0, wsem1)
        wid = lax.axis_index("s") * NC + lax.axis_index("c")

        pltpu.sync_copy(ids_hbm.at[wid], idx_v)

        def gather_copy(j, b):
            return pltpu.make_async_copy(
                table_hbm.at[idx_v.at[j]], rows_v.at[b], gsem[b])

        def write_copy(j, b):
            row0 = (wid * n_steps + j) * ROWS
            return pltpu.make_async_copy(
                rows_v.at[b], out_hbm.at[pl.ds(row0, ROWS)], wsem[b])

        for b in range(NBUF):
            gather_copy(b, b).start()

        def group(i0, carry):
            for b in range(NBUF):
                j = i0 * NBUF + b
                gather_copy(j, b).wait()
                write_copy(j, b).start()
                jn = j + NBUF

                @pl.when(jn < n_steps)
                def _():
                    write_copy(j, b).wait()
                    gather_copy(jn, b).start()

            return carry

        lax.fori_loop(0, n_groups, group, 0)

        for b in range(NBUF):
            write_copy(n_steps - NBUF + b, b).wait()

    return gather_kernel


def kernel(ids, emb_weight):
    batch, length = ids.shape
    V, D = emb_weight.shape
    B = batch * length
    NW = 32
    ids_flat = ids.reshape(NW, B // (NW * ROWS), ROWS)
    out = _make_gather(V, D, B)(emb_weight, ids_flat)
    return out.reshape(batch, length, D)
